# baseline (speedup 1.0000x reference)
import functools

import jax
import jax.numpy as jnp
from jax import lax
from jax.experimental import pallas as pl
from jax.experimental.pallas import tpu as pltpu
from jax.experimental.pallas import tpu_sc as plsc

NC = 2
NS = 16
NW = NC * NS
LANE = 128
CH = 2560
NR = CH // LANE
MAGIC = 12582912.0


def _make_tc_project(B, Npad, C, H, W):
    rows = (B * Npad) // LANE
    rows_per_batch = Npad // LANE
    zero_idx = B * H * W
    hW = float(0.5 * W)
    hH = float(0.5 * H)

    def body(x0_ref, x1_ref, z_ref, idx_ref, valid_ref):
        zf = z_ref[...]
        a = x0_ref[...].astype(jnp.bfloat16).astype(jnp.float32)
        b = x1_ref[...].astype(jnp.bfloat16).astype(jnp.float32)
        z = zf.astype(jnp.bfloat16).astype(jnp.float32)
        uf = (a * hW + z * hW) / z
        vf = (b * hH + z * hH) / z
        ur = (uf + MAGIC) - MAGIC
        vr = (vf + MAGIC) - MAGIC
        cond = ((ur > 0.0) & (ur < W) & (vr > 0.0) & (vr < H) & (zf > 0.0))
        ui = ur.astype(jnp.int32)
        vi = vr.astype(jnp.int32)
        rid = lax.broadcasted_iota(jnp.int32, (rows, LANE), 0)
        tbase = (rid // rows_per_batch) * (H * W)
        ind = ui + vi * H + tbase
        idx_ref[...] = jnp.where(cond, ind, zero_idx)
        valid_ref[...] = jnp.where(cond, 1, 0).astype(jnp.int32)

    return pl.pallas_call(
        body,
        out_shape=(
            jax.ShapeDtypeStruct((rows, LANE), jnp.int32),
            jax.ShapeDtypeStruct((rows, LANE), jnp.int32),
        ),
    )


def _make_sc_gather(B, Npad, C, n_table_rows):
    PW = (B * Npad) // NW
    CHUNKS = PW // CH
    ROWS_W = PW // LANE
    mesh = plsc.VectorSubcoreMesh(core_axis_name="c", subcore_axis_name="s")

    @functools.partial(
        pl.kernel,
        mesh=mesh,
        out_type=jax.ShapeDtypeStruct((B * Npad, C), jnp.float32),
        scratch_types=[
            pltpu.VMEM((NR, LANE), jnp.int32),
            pltpu.VMEM((CH, C), jnp.float32),
            pltpu.SemaphoreType.DMA,
        ],
        compiler_params=pltpu.CompilerParams(use_tc_tiling_on_sc=False),
    )
    def sc_kernel(tab_hbm, idx_hbm, feat_hbm, idxv, featv, sem):
        wid = lax.axis_index("s") * NC + lax.axis_index("c")
        row0 = wid * ROWS_W

        def chunk_body(ci, carry):
            r_off = row0 + ci * NR
            pltpu.sync_copy(idx_hbm.at[pl.ds(r_off, NR)], idxv)
            copies = [
                pltpu.async_copy(tab_hbm.at[idxv.at[r]],
                                 featv.at[pl.ds(r * LANE, LANE)], sem)
                for r in range(NR)
            ]
            for cp in copies:
                cp.wait()
            pltpu.sync_copy(featv, feat_hbm.at[pl.ds(r_off * LANE, CH)])
            return carry

        lax.fori_loop(0, CHUNKS, chunk_body, 0)

    return sc_kernel


def kernel(x, img):
    B, N, _ = x.shape
    _, C, H, W = img.shape

    WPB = NW // B
    per_worker = -(-N // WPB)
    per_worker = -(-per_worker // CH) * CH
    Npad = per_worker * WPB

    xp = jnp.pad(x, ((0, 0), (0, Npad - N), (0, 0)))
    rows = (B * Npad) // LANE
    x0 = xp[..., 0].reshape(rows, LANE)
    x1 = xp[..., 1].reshape(rows, LANE)
    x2 = xp[..., 2].reshape(rows, LANE)

    imgT = jnp.swapaxes(img.reshape(B, C, H * W), 1, 2).reshape(B * H * W, C)
    table = jnp.concatenate([imgT, jnp.zeros((8, C), imgT.dtype)], axis=0)

    idx, valid2d = _make_tc_project(B, Npad, C, H, W)(x0, x1, x2)
    feat_pad = _make_sc_gather(B, Npad, C, table.shape[0])(table, idx)

    feat = feat_pad.reshape(B, Npad, C)[:, :N]
    valid = valid2d.reshape(B, Npad)[:, :N, None].astype(jnp.int64)
    return (feat, valid)
